# initial kernel scaffold (unmeasured)
import jax
import jax.numpy as jnp
from jax import lax
from jax.experimental import pallas as pl
from jax.experimental.pallas import tpu as pltpu


def kernel(
    x,
):
    def body(*refs):
        pass

    out_shape = jax.ShapeDtypeStruct(..., jnp.float32)
    return pl.pallas_call(body, out_shape=out_shape)(...)


# baseline (device time: 12326 ns/iter reference)
import jax
import jax.numpy as jnp
from jax import lax
from jax.experimental import pallas as pl
from jax.experimental.pallas import tpu as pltpu

N_DEV = 8


def kernel(x):
    m_per, n = x.shape

    def body(x_ref, out_ref, comm_ref, send_sems, recv_sems):
        me = lax.axis_index("i")

        logx = jnp.log(x_ref[...])
        row = lax.broadcasted_iota(jnp.int32, (m_per, m_per), 0)
        col = lax.broadcasted_iota(jnp.int32, (m_per, m_per), 1)
        tril = (row >= col).astype(jnp.float32)
        cum = jnp.dot(tril, logx, preferred_element_type=jnp.float32)

        comm_ref[me] = cum[m_per - 1 :, :]

        for d in range(1, N_DEV):
            target = lax.rem(me + d, N_DEV)
            rdma = pltpu.make_async_remote_copy(
                src_ref=comm_ref.at[me],
                dst_ref=comm_ref.at[me],
                send_sem=send_sems.at[d],
                recv_sem=recv_sems.at[d],
                device_id=(target,),
                device_id_type=pl.DeviceIdType.MESH,
            )
            rdma.start()

        for d in range(1, N_DEV):
            src_pos = lax.rem(me - d + N_DEV, N_DEV)
            recv = pltpu.make_async_remote_copy(
                src_ref=comm_ref.at[me],
                dst_ref=comm_ref.at[src_pos],
                send_sem=send_sems.at[d],
                recv_sem=recv_sems.at[d],
                device_id=(src_pos,),
                device_id_type=pl.DeviceIdType.MESH,
            )
            recv.wait_recv()

        totals = comm_ref[...]
        slot = lax.broadcasted_iota(jnp.int32, (N_DEV, 1, n), 0)
        offset = jnp.sum(
            jnp.where(slot < me, totals, 0.0), axis=0
        )
        out_ref[...] = jnp.exp(cum + offset)

        for d in range(1, N_DEV):
            target = lax.rem(me + d, N_DEV)
            send = pltpu.make_async_remote_copy(
                src_ref=comm_ref.at[me],
                dst_ref=comm_ref.at[me],
                send_sem=send_sems.at[d],
                recv_sem=recv_sems.at[d],
                device_id=(target,),
                device_id_type=pl.DeviceIdType.MESH,
            )
            send.wait_send()

    return pl.pallas_call(
        body,
        out_shape=jax.ShapeDtypeStruct((m_per, n), jnp.float32),
        in_specs=[pl.BlockSpec(memory_space=pltpu.VMEM)],
        out_specs=pl.BlockSpec(memory_space=pltpu.VMEM),
        scratch_shapes=[
            pltpu.VMEM((N_DEV, 1, n), jnp.float32),
            pltpu.SemaphoreType.DMA((N_DEV,)),
            pltpu.SemaphoreType.DMA((N_DEV,)),
        ],
    )(x)


# device time: 8270 ns/iter; 1.4904x vs baseline; 1.4904x over previous
import jax
import jax.numpy as jnp
from jax import lax
from jax.experimental import pallas as pl
from jax.experimental.pallas import tpu as pltpu

N_DEV = 8


def kernel(x):
    m_per, n = x.shape

    def body(x_ref, out_ref, comm_ref, send_sems, recv_sems):
        me = lax.axis_index("i")

        barrier_sem = pltpu.get_barrier_semaphore()
        for d in range(1, N_DEV):
            target = lax.rem(me + d, N_DEV)
            pl.semaphore_signal(
                barrier_sem,
                inc=1,
                device_id=(target,),
                device_id_type=pl.DeviceIdType.MESH,
            )

        logx = jnp.log(x_ref[...])
        total = jnp.sum(logx, axis=0, keepdims=True)

        pl.semaphore_wait(barrier_sem, N_DEV - 1)
        comm_ref[me] = total

        for d in range(1, N_DEV):
            target = lax.rem(me + d, N_DEV)
            rdma = pltpu.make_async_remote_copy(
                src_ref=comm_ref.at[me],
                dst_ref=comm_ref.at[me],
                send_sem=send_sems.at[d],
                recv_sem=recv_sems.at[d],
                device_id=(target,),
                device_id_type=pl.DeviceIdType.MESH,
            )
            rdma.start()

        row = lax.broadcasted_iota(jnp.int32, (m_per, m_per), 0)
        col = lax.broadcasted_iota(jnp.int32, (m_per, m_per), 1)
        tril = (row >= col).astype(jnp.float32)
        cum = jnp.dot(tril, logx, preferred_element_type=jnp.float32)

        for d in range(1, N_DEV):
            src_pos = lax.rem(me - d + N_DEV, N_DEV)
            recv = pltpu.make_async_remote_copy(
                src_ref=comm_ref.at[me],
                dst_ref=comm_ref.at[src_pos],
                send_sem=send_sems.at[d],
                recv_sem=recv_sems.at[d],
                device_id=(src_pos,),
                device_id_type=pl.DeviceIdType.MESH,
            )
            recv.wait_recv()

        totals = comm_ref[...]
        slot = lax.broadcasted_iota(jnp.int32, (N_DEV, 1, n), 0)
        offset = jnp.sum(
            jnp.where(slot < me, totals, 0.0), axis=0
        )
        out_ref[...] = jnp.exp(cum + offset)

        for d in range(1, N_DEV):
            target = lax.rem(me + d, N_DEV)
            send = pltpu.make_async_remote_copy(
                src_ref=comm_ref.at[me],
                dst_ref=comm_ref.at[me],
                send_sem=send_sems.at[d],
                recv_sem=recv_sems.at[d],
                device_id=(target,),
                device_id_type=pl.DeviceIdType.MESH,
            )
            send.wait_send()

    return pl.pallas_call(
        body,
        out_shape=jax.ShapeDtypeStruct((m_per, n), jnp.float32),
        in_specs=[pl.BlockSpec(memory_space=pltpu.VMEM)],
        out_specs=pl.BlockSpec(memory_space=pltpu.VMEM),
        scratch_shapes=[
            pltpu.VMEM((N_DEV, 1, n), jnp.float32),
            pltpu.SemaphoreType.DMA((N_DEV,)),
            pltpu.SemaphoreType.DMA((N_DEV,)),
        ],
        compiler_params=pltpu.CompilerParams(collective_id=0),
    )(x)


# device time: 8096 ns/iter; 1.5225x vs baseline; 1.0215x over previous
import jax
import jax.numpy as jnp
from jax import lax
from jax.experimental import pallas as pl
from jax.experimental.pallas import tpu as pltpu

N_DEV = 8


def kernel(x):
    m_per, n = x.shape

    def body(x_ref, out_ref, comm_ref, send_sems, recv_sems):
        me = lax.axis_index("i")

        barrier_sem = pltpu.get_barrier_semaphore()
        for d in range(1, N_DEV):
            target = lax.rem(me + d, N_DEV)
            pl.semaphore_signal(
                barrier_sem,
                inc=1,
                device_id=(target,),
                device_id_type=pl.DeviceIdType.MESH,
            )

        logx = jnp.log(x_ref[...])
        total = jnp.sum(logx, axis=0, keepdims=True)

        pl.semaphore_wait(barrier_sem, N_DEV - 1)
        comm_ref[me] = total

        for d in range(1, N_DEV):
            target = lax.rem(me + d, N_DEV)
            rdma = pltpu.make_async_remote_copy(
                src_ref=comm_ref.at[me],
                dst_ref=comm_ref.at[me],
                send_sem=send_sems.at[d],
                recv_sem=recv_sems.at[d],
                device_id=(target,),
                device_id_type=pl.DeviceIdType.MESH,
            )
            rdma.start()

        row = lax.broadcasted_iota(jnp.int32, (m_per, m_per), 0)
        col = lax.broadcasted_iota(jnp.int32, (m_per, m_per), 1)
        tril = (row >= col).astype(jnp.bfloat16)
        cum = jnp.dot(
            tril,
            logx.astype(jnp.bfloat16),
            preferred_element_type=jnp.float32,
        )
        z = jnp.exp(cum)

        for d in range(1, N_DEV):
            src_pos = lax.rem(me - d + N_DEV, N_DEV)
            recv = pltpu.make_async_remote_copy(
                src_ref=comm_ref.at[me],
                dst_ref=comm_ref.at[src_pos],
                send_sem=send_sems.at[d],
                recv_sem=recv_sems.at[d],
                device_id=(src_pos,),
                device_id_type=pl.DeviceIdType.MESH,
            )
            recv.wait_recv()

        totals = comm_ref[...]
        slot = lax.broadcasted_iota(jnp.int32, (N_DEV, 1, n), 0)
        offset = jnp.sum(
            jnp.where(slot < me, totals, 0.0), axis=0
        )
        out_ref[...] = z * jnp.exp(offset)

        for d in range(1, N_DEV):
            target = lax.rem(me + d, N_DEV)
            send = pltpu.make_async_remote_copy(
                src_ref=comm_ref.at[me],
                dst_ref=comm_ref.at[me],
                send_sem=send_sems.at[d],
                recv_sem=recv_sems.at[d],
                device_id=(target,),
                device_id_type=pl.DeviceIdType.MESH,
            )
            send.wait_send()

    return pl.pallas_call(
        body,
        out_shape=jax.ShapeDtypeStruct((m_per, n), jnp.float32),
        in_specs=[pl.BlockSpec(memory_space=pltpu.VMEM)],
        out_specs=pl.BlockSpec(memory_space=pltpu.VMEM),
        scratch_shapes=[
            pltpu.VMEM((N_DEV, 1, n), jnp.float32),
            pltpu.SemaphoreType.DMA((N_DEV,)),
            pltpu.SemaphoreType.DMA((N_DEV,)),
        ],
        compiler_params=pltpu.CompilerParams(collective_id=0),
    )(x)
